# baseline (speedup 1.0000x reference)
import jax
import jax.numpy as jnp
from jax.experimental import pallas as pl
from jax.experimental.pallas import tpu as pltpu

_N = 4096
_F = 128
_H = 128
_C = 32
_BS = 512
_NS = _N // _BS
_T = 0.2

_HP = jax.lax.Precision.HIGHEST


def _log_softmax_t(z):
    m = jnp.max(z, axis=0, keepdims=True)
    zm = z - m
    lse = jnp.log(jnp.sum(jnp.exp(zm), axis=0, keepdims=True))
    return zm - lse


def _mm(a, b, precision=None):
    return jax.lax.dot_general(a, b, (((1,), (0,)), ((), ())),
                               precision=precision,
                               preferred_element_type=jnp.float32)


def _fused(adj_ref, xt_ref, w1t_ref, b1t_ref, w2t_ref, b2t_ref,
           wm1t_ref, bm1t_ref, wm2t_ref, bm2t_ref, out_ref,
           adjb_ref, deg_ref, u1_ref, xw1_ref, y1_ref):
    i = pl.program_id(0)

    @pl.when(i == 0)
    def _init():
        adjb_ref[...] = jnp.zeros((_N, _N), jnp.bfloat16)
        u1_ref[...] = jnp.zeros((_H, _N), jnp.bfloat16)
        y1_ref[...] = jnp.zeros((_H, _N), jnp.float32)
        xw1_ref[...] = _mm(w1t_ref[...], xt_ref[...], _HP)

    @pl.when((i >= 1) & (i <= _NS))
    def _ingest():
        j = i - 1
        sl = pl.ds(j * _BS, _BS)
        blk = adj_ref[...]
        part = jnp.sum(blk, axis=0, keepdims=True)
        deg_ref[:, sl] = part
        dinv = jax.lax.rsqrt(part + 1.0)
        u1_ref[:, sl] = (dinv * xw1_ref[:, sl]).astype(jnp.bfloat16)

        @pl.when(j > 0)
        def _rows():
            y1_ref[...] += _mm(u1_ref[:, sl], adjb_ref[pl.ds(j * _BS, _BS), :])

        adjb_ref[:, sl] = blk.astype(jnp.bfloat16)

        @pl.when(j < _NS // 2)
        def _cols_half():
            y1_ref[:, sl] += _mm(u1_ref[:, : _N // 2],
                                 adjb_ref[: _N // 2, sl])

        @pl.when(j >= _NS // 2)
        def _cols_full():
            y1_ref[:, sl] += _mm(u1_ref[...], adjb_ref[:, sl])

    @pl.when(i == _NS + 1)
    def _tail():
        dinv = jax.lax.rsqrt(deg_ref[...] + 1.0)
        d2 = dinv * dinv
        g1 = dinv * y1_ref[...] + d2 * xw1_ref[...] + b1t_ref[...]
        h = jnp.maximum(g1, 0.0)
        xw2 = _mm(w2t_ref[...], h, _HP)
        u2 = (dinv * xw2).astype(jnp.bfloat16)
        y2 = _mm(u2, adjb_ref[...])
        g2 = dinv * y2 + d2 * xw2 + b2t_ref[...]
        s_pred = _log_softmax_t(g2 / _T)

        t1 = jnp.maximum(_mm(wm1t_ref[...], xt_ref[...], _HP)
                         + bm1t_ref[...], 0.0)
        f_logits = _mm(wm2t_ref[...], t1, _HP) + bm2t_ref[...]
        f_pred = _log_softmax_t(f_logits / _T)

        out_ref[...] = (f_pred + s_pred) * 0.5


def kernel(x, adj, W1, b1, W2, b2, Wm1, bm1, Wm2, bm2):
    def full(r, c):
        return pl.BlockSpec((r, c), lambda i: (0, 0))

    out_t = pl.pallas_call(
        _fused,
        grid=(_NS + 2,),
        in_specs=[
            pl.BlockSpec((_N, _BS), lambda i: (0, jnp.clip(i - 1, 0, _NS - 1))),
            full(_F, _N),
            full(_H, _F), full(_H, 1),
            full(_C, _H), full(_C, 1),
            full(_H, _F), full(_H, 1),
            full(_C, _H), full(_C, 1),
        ],
        out_specs=full(_C, _N),
        out_shape=jax.ShapeDtypeStruct((_C, _N), jnp.float32),
        scratch_shapes=[
            pltpu.VMEM((_N, _N), jnp.bfloat16),
            pltpu.VMEM((1, _N), jnp.float32),
            pltpu.VMEM((_H, _N), jnp.bfloat16),
            pltpu.VMEM((_H, _N), jnp.float32),
            pltpu.VMEM((_H, _N), jnp.float32),
        ],
        compiler_params=pltpu.CompilerParams(
            dimension_semantics=("arbitrary",),
            vmem_limit_bytes=128 * 1024 * 1024,
        ),
    )(adj, x.T, W1.T, b1.reshape(_H, 1), W2.T, b2.reshape(_C, 1),
      Wm1.T, bm1.reshape(_H, 1), Wm2.T, bm2.reshape(_C, 1))
    return out_t.T


# speedup vs baseline: 1.0882x; 1.0882x over previous
import jax
import jax.numpy as jnp
from jax.experimental import pallas as pl
from jax.experimental.pallas import tpu as pltpu

_N = 4096
_F = 128
_H = 128
_C = 32
_BS = 512
_NS = _N // _BS
_T = 0.2

_HP = jax.lax.Precision.HIGHEST


def _log_softmax_t(z):
    m = jnp.max(z, axis=0, keepdims=True)
    zm = z - m
    lse = jnp.log(jnp.sum(jnp.exp(zm), axis=0, keepdims=True))
    return zm - lse


def _mm(a, b, precision=None):
    return jax.lax.dot_general(a, b, (((1,), (0,)), ((), ())),
                               precision=precision,
                               preferred_element_type=jnp.float32)


def _fused(adj_ref, xt_ref, w1t_ref, b1t_ref, w2t_ref, b2t_ref,
           wm1t_ref, bm1t_ref, wm2t_ref, bm2t_ref, out_ref,
           adjb_ref, deg_ref, u1_ref, xw1_ref, y1_ref):
    i = pl.program_id(0)

    @pl.when(i < _NS)
    def _ingest():
        @pl.when(i == 0)
        def _init():
            u1_ref[...] = jnp.zeros((_H, _N), jnp.bfloat16)
            xw1_ref[...] = _mm(w1t_ref[...], xt_ref[...],
                               _HP).astype(jnp.bfloat16)

        sl = pl.ds(i * _BS, _BS)
        blk = adj_ref[...]
        part = jnp.sum(blk, axis=0, keepdims=True)
        deg_ref[:, sl] = part
        dinv = jax.lax.rsqrt(part + 1.0)
        u1_ref[:, sl] = (dinv * xw1_ref[:, sl].astype(jnp.float32)
                         ).astype(jnp.bfloat16)

        @pl.when((i > 0) & (i <= _NS // 2))
        def _rows_half():
            yc = _mm(u1_ref[:, sl], adjb_ref[pl.ds(i * _BS, _BS), : _N // 2])
            col = jax.lax.broadcasted_iota(jnp.int32, (1, _N // 2), 1)
            y1_ref[:, : _N // 2] += jnp.where(col < i * _BS, yc, 0.0)

        @pl.when(i > _NS // 2)
        def _rows_full():
            yc = _mm(u1_ref[:, sl], adjb_ref[pl.ds(i * _BS, _BS), :])
            col = jax.lax.broadcasted_iota(jnp.int32, (1, _N), 1)
            y1_ref[...] += jnp.where(col < i * _BS, yc, 0.0)

        adjb_ref[:, sl] = blk.astype(jnp.bfloat16)

        @pl.when(i < _NS // 2)
        def _cols_half():
            y1_ref[:, sl] = _mm(u1_ref[:, : _N // 2], adjb_ref[: _N // 2, sl])

        @pl.when(i >= _NS // 2)
        def _cols_full():
            y1_ref[:, sl] = _mm(u1_ref[...], adjb_ref[:, sl])

    @pl.when(i == _NS)
    def _tail():
        dinv = jax.lax.rsqrt(deg_ref[...] + 1.0)
        d2 = dinv * dinv
        g1 = (dinv * y1_ref[...] + d2 * xw1_ref[...].astype(jnp.float32)
              + b1t_ref[...])
        h = jnp.maximum(g1, 0.0)
        xw2 = _mm(w2t_ref[...], h, _HP)
        u2 = (dinv * xw2).astype(jnp.bfloat16)
        y2 = _mm(u2, adjb_ref[...])
        g2 = dinv * y2 + d2 * xw2 + b2t_ref[...]
        s_pred = _log_softmax_t(g2 / _T)

        t1 = jnp.maximum(_mm(wm1t_ref[...], xt_ref[...], _HP)
                         + bm1t_ref[...], 0.0)
        f_logits = _mm(wm2t_ref[...], t1, _HP) + bm2t_ref[...]
        f_pred = _log_softmax_t(f_logits / _T)

        out_ref[...] = (f_pred + s_pred) * 0.5


def kernel(x, adj, W1, b1, W2, b2, Wm1, bm1, Wm2, bm2):
    def full(r, c):
        return pl.BlockSpec((r, c), lambda i: (0, 0))

    out_t = pl.pallas_call(
        _fused,
        grid=(_NS + 1,),
        in_specs=[
            pl.BlockSpec((_N, _BS), lambda i: (0, jnp.minimum(i, _NS - 1))),
            full(_F, _N),
            full(_H, _F), full(_H, 1),
            full(_C, _H), full(_C, 1),
            full(_H, _F), full(_H, 1),
            full(_C, _H), full(_C, 1),
        ],
        out_specs=full(_C, _N),
        out_shape=jax.ShapeDtypeStruct((_C, _N), jnp.float32),
        scratch_shapes=[
            pltpu.VMEM((_N, _N), jnp.bfloat16),
            pltpu.VMEM((1, _N), jnp.float32),
            pltpu.VMEM((_H, _N), jnp.bfloat16),
            pltpu.VMEM((_H, _N), jnp.bfloat16),
            pltpu.VMEM((_H, _N), jnp.float32),
        ],
        compiler_params=pltpu.CompilerParams(
            dimension_semantics=("arbitrary",),
            vmem_limit_bytes=128 * 1024 * 1024,
        ),
    )(adj, x.T, W1.T, b1.reshape(_H, 1), W2.T, b2.reshape(_C, 1),
      Wm1.T, bm1.reshape(_H, 1), Wm2.T, bm2.reshape(_C, 1))
    return out_t.T


# speedup vs baseline: 1.1547x; 1.0611x over previous
import jax
import jax.numpy as jnp
from jax.experimental import pallas as pl
from jax.experimental.pallas import tpu as pltpu

_N = 4096
_F = 128
_H = 128
_C = 32
_BR = 512
_NR = _N // _BR
_BC = 4096
_NC = _N // _BC
_T = 0.2

_HP = jax.lax.Precision.HIGHEST


def _log_softmax_t(z):
    m = jnp.max(z, axis=0, keepdims=True)
    zm = z - m
    lse = jnp.log(jnp.sum(jnp.exp(zm), axis=0, keepdims=True))
    return zm - lse


def _mm(a, b, precision=None):
    return jax.lax.dot_general(a, b, (((1,), (0,)), ((), ())),
                               precision=precision,
                               preferred_element_type=jnp.float32)


def _fused(adj_ref, xt_ref, w1t_ref, b1t_ref, w2t_ref, b2t_ref,
           wm1t_ref, bm1t_ref, wm2t_ref, bm2t_ref, out_ref,
           adjb_ref, deg_ref, u1_ref, xw1_ref, u2_ref, xw2_ref, fp_ref):
    i = pl.program_id(0)

    @pl.when(i < _NR)
    def _ingest():
        blk = adj_ref[...]
        adjb_ref[pl.ds(i * _BR, _BR), :] = blk.astype(jnp.bfloat16)
        part = jnp.sum(blk, axis=0, keepdims=True)

        @pl.when(i == 0)
        def _init():
            deg_ref[...] = part
            xw1_ref[...] = _mm(w1t_ref[...], xt_ref[...], _HP)

        @pl.when(i > 0)
        def _acc():
            deg_ref[...] += part

        @pl.when(i == 1)
        def _mlp():
            t1 = jnp.maximum(_mm(wm1t_ref[...], xt_ref[...], _HP)
                             + bm1t_ref[...], 0.0)
            f_logits = _mm(wm2t_ref[...], t1, _HP) + bm2t_ref[...]
            fp_ref[...] = _log_softmax_t(f_logits / _T)

        @pl.when(i == _NR - 1)
        def _prep1():
            dinv = jax.lax.rsqrt(deg_ref[...] + 1.0)
            u1_ref[...] = (dinv * xw1_ref[...]).astype(jnp.bfloat16)

    @pl.when((i >= _NR) & (i < _NR + _NC))
    def _conv1():
        c = i - _NR
        sl = pl.ds(c * _BC, _BC)
        dinv = jax.lax.rsqrt(deg_ref[:, sl] + 1.0)
        y1 = _mm(u1_ref[...], adjb_ref[:, sl])
        g1 = dinv * y1 + (dinv * dinv) * xw1_ref[:, sl] + b1t_ref[...]
        h = jnp.maximum(g1, 0.0)
        xw2 = _mm(w2t_ref[...], h, _HP)
        xw2_ref[:, sl] = xw2
        u2_ref[:, sl] = (dinv * xw2).astype(jnp.bfloat16)

    @pl.when(i >= _NR + _NC)
    def _conv2():
        c = i - _NR - _NC
        sl = pl.ds(c * _BC, _BC)
        dinv = jax.lax.rsqrt(deg_ref[:, sl] + 1.0)
        y2 = _mm(u2_ref[...], adjb_ref[:, sl])
        g2 = dinv * y2 + (dinv * dinv) * xw2_ref[:, sl] + b2t_ref[...]
        s_pred = _log_softmax_t(g2 / _T)
        out_ref[...] = (fp_ref[:, sl] + s_pred) * 0.5


def kernel(x, adj, W1, b1, W2, b2, Wm1, bm1, Wm2, bm2):
    def full(r, c):
        return pl.BlockSpec((r, c), lambda i: (0, 0))

    out_t = pl.pallas_call(
        _fused,
        grid=(_NR + 2 * _NC,),
        in_specs=[
            pl.BlockSpec((_BR, _N), lambda i: (jnp.minimum(i, _NR - 1), 0)),
            full(_F, _N),
            full(_H, _F), full(_H, 1),
            full(_C, _H), full(_C, 1),
            full(_H, _F), full(_H, 1),
            full(_C, _H), full(_C, 1),
        ],
        out_specs=pl.BlockSpec(
            (_C, _BC), lambda i: (0, jnp.clip(i - _NR - _NC, 0, _NC - 1))),
        out_shape=jax.ShapeDtypeStruct((_C, _N), jnp.float32),
        scratch_shapes=[
            pltpu.VMEM((_N, _N), jnp.bfloat16),
            pltpu.VMEM((1, _N), jnp.float32),
            pltpu.VMEM((_H, _N), jnp.bfloat16),
            pltpu.VMEM((_H, _N), jnp.float32),
            pltpu.VMEM((_C, _N), jnp.bfloat16),
            pltpu.VMEM((_C, _N), jnp.float32),
            pltpu.VMEM((_C, _N), jnp.float32),
        ],
        compiler_params=pltpu.CompilerParams(
            dimension_semantics=("arbitrary",),
            vmem_limit_bytes=128 * 1024 * 1024,
        ),
    )(adj, x.T, W1.T, b1.reshape(_H, 1), W2.T, b2.reshape(_C, 1),
      Wm1.T, bm1.reshape(_H, 1), Wm2.T, bm2.reshape(_C, 1))
    return out_t.T
